# baseline (device time: 142908 ns/iter reference)
import numpy as np

import jax
import jax.numpy as jnp
from jax import lax
from jax.experimental import pallas as pl
from jax.experimental.pallas import tpu as pltpu

N_DEV = 32
SQ = 1024
D = 1024
HQ = 8
DH = 128
ROWS = SQ // N_DEV
BLK = 128
SCALE = 0.08838834764831843


def _rope_tables():
    inv = 1.0 / (10000.0 ** (np.arange(0, DH, 2) / DH))
    pos = np.arange(SQ)[:, None] * inv[None, :]
    cos = np.tile(np.repeat(np.cos(pos), 2, axis=-1), (1, HQ)).astype(np.float32)
    sin = np.tile(np.repeat(np.sin(pos), 2, axis=-1), (1, HQ)).astype(np.float32)
    return jnp.asarray(cos), jnp.asarray(sin)


def _qkv(x, Wq, Wk, Wv):
    xb = x[0].astype(jnp.bfloat16)
    cos, sin = _rope_tables()

    def rope(t, scale):
        tf = t.astype(jnp.float32)
        t2 = tf.reshape(SQ, D // 2, 2)
        tr = jnp.stack([-t2[..., 1], t2[..., 0]], axis=-1).reshape(SQ, D)
        return ((tf * cos + tr * sin) * scale).astype(jnp.bfloat16)

    q = rope(xb @ Wq.astype(jnp.bfloat16), SCALE)
    k = rope(xb @ Wk.astype(jnp.bfloat16), 1.0)
    v = xb @ Wv.astype(jnp.bfloat16)
    return q, k, v


def _fused_body(q_ref, k_ref, v_ref, wo_ref, out_ref,
                partial_ref, recv_ref, acc_ref, red_ref,
                rs_send, rs_recv, ag_send, ag_recv):
    me = lax.axis_index("i")

    barrier = pltpu.get_barrier_semaphore()
    for k in range(1, N_DEV):
        peer = lax.rem(me + k, N_DEV)
        pl.semaphore_signal(barrier, inc=1, device_id=(peer,),
                            device_id_type=pl.DeviceIdType.MESH)
    pl.semaphore_wait(barrier, N_DEV - 1)

    for b in range(SQ // BLK):
        rows = pl.ds(b * BLK, BLK)
        pblk = jnp.zeros((BLK, D), jnp.float32)
        for h in range(HQ):
            cols = slice(h * DH, (h + 1) * DH)
            qb = q_ref[rows, cols]
            s = lax.dot_general(
                qb, k_ref[:, cols], (((1,), (1,)), ((), ())),
                preferred_element_type=jnp.float32,
            )
            e = jnp.exp(s)
            r = 1.0 / jnp.sum(e, axis=1, keepdims=True)
            c = jnp.dot(e.astype(jnp.bfloat16), v_ref[:, cols],
                        preferred_element_type=jnp.float32)
            c = (c * r).astype(jnp.bfloat16)
            pblk = pblk + jnp.dot(c, wo_ref[cols, :],
                                  preferred_element_type=jnp.float32)
        partial_ref[rows, :] = pblk.astype(jnp.bfloat16)

        for d in range(b * (BLK // ROWS), (b + 1) * (BLK // ROWS)):
            kk = lax.rem(jnp.int32(d) - me + N_DEV, N_DEV)
            desc = pltpu.make_async_remote_copy(
                src_ref=partial_ref.at[pl.ds(d * ROWS, ROWS), :],
                dst_ref=recv_ref.at[kk],
                send_sem=rs_send.at[kk],
                recv_sem=rs_recv.at[kk],
                device_id=(jnp.int32(d),),
                device_id_type=pl.DeviceIdType.MESH,
            )

            @pl.when(jnp.int32(d) != me)
            def _():
                desc.start()

    acc_ref[...] = partial_ref[pl.ds(me * ROWS, ROWS), :].astype(jnp.float32)
    for k in range(1, N_DEV):
        rcv = pltpu.make_async_remote_copy(
            src_ref=partial_ref.at[pl.ds(0, ROWS), :],
            dst_ref=recv_ref.at[k],
            send_sem=rs_send.at[k],
            recv_sem=rs_recv.at[k],
            device_id=(me,),
            device_id_type=pl.DeviceIdType.MESH,
        )
        rcv.wait_recv()
        acc_ref[...] += recv_ref[k].astype(jnp.float32)

    red_ref[...] = acc_ref[...].astype(jnp.bfloat16)
    ag = []
    for k in range(1, N_DEV):
        dst = lax.rem(me + k, N_DEV)
        d = pltpu.make_async_remote_copy(
            src_ref=red_ref,
            dst_ref=out_ref.at[pl.ds(me * ROWS, ROWS), :],
            send_sem=ag_send.at[k],
            recv_sem=ag_recv.at[k],
            device_id=(dst,),
            device_id_type=pl.DeviceIdType.MESH,
        )
        d.start()
        ag.append(d)

    out_ref[pl.ds(me * ROWS, ROWS), :] = red_ref[...]

    for k in range(1, N_DEV):
        src = lax.rem(me - k + N_DEV, N_DEV)
        recv = pltpu.make_async_remote_copy(
            src_ref=red_ref,
            dst_ref=out_ref.at[pl.ds(src * ROWS, ROWS), :],
            send_sem=ag_send.at[k],
            recv_sem=ag_recv.at[k],
            device_id=(src,),
            device_id_type=pl.DeviceIdType.MESH,
        )
        recv.wait_recv()

    for k in range(1, N_DEV):
        snd = pltpu.make_async_remote_copy(
            src_ref=partial_ref.at[pl.ds(0, ROWS), :],
            dst_ref=recv_ref.at[k],
            send_sem=rs_send.at[k],
            recv_sem=rs_recv.at[k],
            device_id=(me,),
            device_id_type=pl.DeviceIdType.MESH,
        )
        snd.wait_send()
        ag[k - 1].wait_send()


def kernel(x, Wq, Wk, Wv, Wo):
    q, k, v = _qkv(x, Wq, Wk, Wv)
    out = pl.pallas_call(
        _fused_body,
        out_shape=jax.ShapeDtypeStruct((SQ, D), jnp.bfloat16),
        in_specs=[pl.BlockSpec(memory_space=pltpu.VMEM)] * 4,
        out_specs=pl.BlockSpec(memory_space=pltpu.VMEM),
        scratch_shapes=[
            pltpu.VMEM((SQ, D), jnp.bfloat16),
            pltpu.VMEM((N_DEV, ROWS, D), jnp.bfloat16),
            pltpu.VMEM((ROWS, D), jnp.float32),
            pltpu.VMEM((ROWS, D), jnp.bfloat16),
            pltpu.SemaphoreType.DMA((N_DEV,)),
            pltpu.SemaphoreType.DMA((N_DEV,)),
            pltpu.SemaphoreType.DMA((N_DEV,)),
            pltpu.SemaphoreType.DMA((N_DEV,)),
        ],
        compiler_params=pltpu.CompilerParams(collective_id=0),
    )(q, k, v, Wo.astype(jnp.bfloat16))
    return out[None, :, :]


# device time: 124352 ns/iter; 1.1492x vs baseline; 1.1492x over previous
import numpy as np

import jax
import jax.numpy as jnp
from jax import lax
from jax.experimental import pallas as pl
from jax.experimental.pallas import tpu as pltpu

N_DEV = 32
SQ = 1024
D = 1024
HQ = 8
DH = 128
ROWS = SQ // N_DEV
BLK = 128
SCALE = 0.08838834764831843


def _tables():
    inv = 1.0 / (10000.0 ** (np.arange(0, DH, 2) / DH))
    pos = np.arange(SQ)[:, None] * inv[None, :]
    cos = np.repeat(np.cos(pos), 2, axis=-1)
    sin = np.repeat(np.sin(pos), 2, axis=-1)
    rot = np.zeros((DH, DH))
    rot[np.arange(1, DH, 2), np.arange(0, DH, 2)] = -1.0
    rot[np.arange(0, DH, 2), np.arange(1, DH, 2)] = 1.0
    bf = jnp.bfloat16
    return (
        jnp.asarray(cos, bf), jnp.asarray(sin, bf),
        jnp.asarray(cos * SCALE, bf), jnp.asarray(sin * SCALE, bf),
        jnp.asarray(rot, bf),
    )


def _fused_body(x_ref, wq_ref, wk_ref, wv_ref, wo_ref,
                cos_ref, sin_ref, qcos_ref, qsin_ref, rot_ref, out_ref,
                q3, k3, v3, partial_ref, recv_ref, acc_ref, red_ref,
                rs_send, rs_recv, ag_send, ag_recv):
    me = lax.axis_index("i")

    barrier = pltpu.get_barrier_semaphore()
    for k in range(1, N_DEV):
        peer = lax.rem(me + k, N_DEV)
        pl.semaphore_signal(barrier, inc=1, device_id=(peer,),
                            device_id_type=pl.DeviceIdType.MESH)
    pl.semaphore_wait(barrier, N_DEV - 1)

    x = x_ref[...]
    f32 = jnp.float32
    bf = jnp.bfloat16
    for h in range(HQ):
        cols = slice(h * DH, (h + 1) * DH)
        qraw = jnp.dot(x, wq_ref[:, cols],
                       preferred_element_type=f32).astype(bf)
        q3[h] = (qraw * qcos_ref[...]
                 + jnp.dot(qraw, rot_ref[...],
                           preferred_element_type=f32).astype(bf)
                 * qsin_ref[...])
        kraw = jnp.dot(x, wk_ref[:, cols],
                       preferred_element_type=f32).astype(bf)
        k3[h] = (kraw * cos_ref[...]
                 + jnp.dot(kraw, rot_ref[...],
                           preferred_element_type=f32).astype(bf)
                 * sin_ref[...])
        v3[h] = jnp.dot(x, wv_ref[:, cols],
                        preferred_element_type=f32).astype(bf)

    for b in range(SQ // BLK):
        rows = pl.ds(b * BLK, BLK)
        pblk = jnp.zeros((BLK, D), jnp.float32)
        for h in range(HQ):
            qb = q3[h, rows, :]
            s = lax.dot_general(
                qb, k3[h], (((1,), (1,)), ((), ())),
                preferred_element_type=jnp.float32,
            )
            e = jnp.exp(s)
            r = 1.0 / jnp.sum(e, axis=1, keepdims=True)
            c = jnp.dot(e.astype(jnp.bfloat16), v3[h],
                        preferred_element_type=jnp.float32)
            c = (c * r).astype(jnp.bfloat16)
            pblk = pblk + jnp.dot(c, wo_ref[h],
                                  preferred_element_type=jnp.float32)
        partial_ref[rows, :] = pblk.astype(jnp.bfloat16)

        for d in range(b * (BLK // ROWS), (b + 1) * (BLK // ROWS)):
            kk = lax.rem(jnp.int32(d) - me + N_DEV, N_DEV)
            desc = pltpu.make_async_remote_copy(
                src_ref=partial_ref.at[pl.ds(d * ROWS, ROWS), :],
                dst_ref=recv_ref.at[kk],
                send_sem=rs_send.at[kk],
                recv_sem=rs_recv.at[kk],
                device_id=(jnp.int32(d),),
                device_id_type=pl.DeviceIdType.MESH,
            )

            @pl.when(jnp.int32(d) != me)
            def _():
                desc.start()

    acc_ref[...] = partial_ref[pl.ds(me * ROWS, ROWS), :].astype(jnp.float32)
    for k in range(1, N_DEV):
        rcv = pltpu.make_async_remote_copy(
            src_ref=partial_ref.at[pl.ds(0, ROWS), :],
            dst_ref=recv_ref.at[k],
            send_sem=rs_send.at[k],
            recv_sem=rs_recv.at[k],
            device_id=(me,),
            device_id_type=pl.DeviceIdType.MESH,
        )
        rcv.wait_recv()
        acc_ref[...] += recv_ref[k].astype(jnp.float32)

    red_ref[...] = acc_ref[...].astype(jnp.bfloat16)
    ag = []
    for k in range(1, N_DEV):
        dst = lax.rem(me + k, N_DEV)
        d = pltpu.make_async_remote_copy(
            src_ref=red_ref,
            dst_ref=out_ref.at[pl.ds(me * ROWS, ROWS), :],
            send_sem=ag_send.at[k],
            recv_sem=ag_recv.at[k],
            device_id=(dst,),
            device_id_type=pl.DeviceIdType.MESH,
        )
        d.start()
        ag.append(d)

    out_ref[pl.ds(me * ROWS, ROWS), :] = red_ref[...]

    for k in range(1, N_DEV):
        src = lax.rem(me - k + N_DEV, N_DEV)
        recv = pltpu.make_async_remote_copy(
            src_ref=red_ref,
            dst_ref=out_ref.at[pl.ds(src * ROWS, ROWS), :],
            send_sem=ag_send.at[k],
            recv_sem=ag_recv.at[k],
            device_id=(src,),
            device_id_type=pl.DeviceIdType.MESH,
        )
        recv.wait_recv()

    for k in range(1, N_DEV):
        snd = pltpu.make_async_remote_copy(
            src_ref=partial_ref.at[pl.ds(0, ROWS), :],
            dst_ref=recv_ref.at[k],
            send_sem=rs_send.at[k],
            recv_sem=rs_recv.at[k],
            device_id=(me,),
            device_id_type=pl.DeviceIdType.MESH,
        )
        snd.wait_send()
        ag[k - 1].wait_send()


def kernel(x, Wq, Wk, Wv, Wo):
    bf = jnp.bfloat16
    cos, sin, qcos, qsin, rot = _tables()
    out = pl.pallas_call(
        _fused_body,
        out_shape=jax.ShapeDtypeStruct((SQ, D), bf),
        in_specs=[pl.BlockSpec(memory_space=pltpu.VMEM)] * 10,
        out_specs=pl.BlockSpec(memory_space=pltpu.VMEM),
        scratch_shapes=[
            pltpu.VMEM((HQ, SQ, DH), bf),
            pltpu.VMEM((HQ, SQ, DH), bf),
            pltpu.VMEM((HQ, SQ, DH), bf),
            pltpu.VMEM((SQ, D), bf),
            pltpu.VMEM((N_DEV, ROWS, D), bf),
            pltpu.VMEM((ROWS, D), jnp.float32),
            pltpu.VMEM((ROWS, D), bf),
            pltpu.SemaphoreType.DMA((N_DEV,)),
            pltpu.SemaphoreType.DMA((N_DEV,)),
            pltpu.SemaphoreType.DMA((N_DEV,)),
            pltpu.SemaphoreType.DMA((N_DEV,)),
        ],
        compiler_params=pltpu.CompilerParams(collective_id=0),
    )(x[0].astype(bf), Wq.astype(bf), Wk.astype(bf), Wv.astype(bf),
      Wo.astype(bf).reshape(HQ, DH, D), cos, sin, qcos, qsin, rot)
    return out[None, :, :]


# device time: 61617 ns/iter; 2.3193x vs baseline; 2.0181x over previous
import numpy as np

import jax
import jax.numpy as jnp
from jax import lax
from jax.experimental import pallas as pl
from jax.experimental.pallas import tpu as pltpu

N_DEV = 32
SQ = 1024
D = 1024
HQ = 8
DH = 128
ROWS = SQ // N_DEV
BLK = 128
SCALE = 0.08838834764831843
_COMM = True


def _tables():
    inv = 1.0 / (10000.0 ** (np.arange(0, DH, 2) / DH))
    pos = np.arange(SQ)[:, None] * inv[None, :]
    cos = np.repeat(np.cos(pos), 2, axis=-1)
    sin = np.repeat(np.sin(pos), 2, axis=-1)
    rot = np.zeros((DH, DH))
    rot[np.arange(1, DH, 2), np.arange(0, DH, 2)] = -1.0
    rot[np.arange(0, DH, 2), np.arange(1, DH, 2)] = 1.0
    bf = jnp.bfloat16
    return (
        jnp.asarray(cos, bf), jnp.asarray(sin, bf),
        jnp.asarray(cos * SCALE, bf), jnp.asarray(sin * SCALE, bf),
        jnp.asarray(rot, bf),
    )


def _fused_body(x_ref, wq_ref, wk_ref, wv_ref, wo_ref,
                cos_ref, sin_ref, qcos_ref, qsin_ref, rot_ref, out_ref,
                q3, k3, v3, partial_ref, recv_ref, acc_ref, red_ref,
                rs_send, rs_recv, ag_send, ag_recv):
    me = lax.axis_index("i")

    if _COMM:
        barrier = pltpu.get_barrier_semaphore()
        for k in range(1, N_DEV):
            peer = lax.rem(me + k, N_DEV)
            pl.semaphore_signal(barrier, inc=1, device_id=(peer,),
                                device_id_type=pl.DeviceIdType.MESH)
        pl.semaphore_wait(barrier, N_DEV - 1)

    x = x_ref[...]
    f32 = jnp.float32
    bf = jnp.bfloat16
    for h in range(HQ):
        cols = slice(h * DH, (h + 1) * DH)
        qraw = jnp.dot(x, wq_ref[:, cols],
                       preferred_element_type=f32).astype(bf)
        q3[h] = (qraw * qcos_ref[...]
                 + jnp.dot(qraw, rot_ref[...],
                           preferred_element_type=f32).astype(bf)
                 * qsin_ref[...])
        kraw = jnp.dot(x, wk_ref[:, cols],
                       preferred_element_type=f32).astype(bf)
        k3[h] = (kraw * cos_ref[...]
                 + jnp.dot(kraw, rot_ref[...],
                           preferred_element_type=f32).astype(bf)
                 * sin_ref[...])
        v3[h] = jnp.dot(x, wv_ref[:, cols],
                        preferred_element_type=f32).astype(bf)

    for b in range(SQ // BLK):
        rows = pl.ds(b * BLK, BLK)
        pblk = jnp.zeros((BLK, D), jnp.float32)
        for h in range(HQ):
            qb = q3[h, rows, :]
            s = lax.dot_general(
                qb, k3[h], (((1,), (1,)), ((), ())),
                preferred_element_type=jnp.float32,
            )
            e = jnp.exp(s)
            r = 1.0 / jnp.sum(e, axis=1, keepdims=True)
            c = jnp.dot(e.astype(jnp.bfloat16), v3[h],
                        preferred_element_type=jnp.float32)
            c = (c * r).astype(jnp.bfloat16)
            pblk = pblk + jnp.dot(c, wo_ref[h],
                                  preferred_element_type=jnp.float32)
        partial_ref[rows, :] = pblk.astype(jnp.bfloat16)

        if not _COMM:
            continue
        for d in range(b * (BLK // ROWS), (b + 1) * (BLK // ROWS)):
            kk = lax.rem(jnp.int32(d) - me + N_DEV, N_DEV)
            desc = pltpu.make_async_remote_copy(
                src_ref=partial_ref.at[pl.ds(d * ROWS, ROWS), :],
                dst_ref=recv_ref.at[kk],
                send_sem=rs_send.at[kk],
                recv_sem=rs_recv.at[kk],
                device_id=(jnp.int32(d),),
                device_id_type=pl.DeviceIdType.MESH,
            )

            @pl.when(jnp.int32(d) != me)
            def _():
                desc.start()

    if not _COMM:
        out_ref[...] = partial_ref[...]
        return

    acc_ref[...] = partial_ref[pl.ds(me * ROWS, ROWS), :].astype(jnp.float32)
    for k in range(1, N_DEV):
        rcv = pltpu.make_async_remote_copy(
            src_ref=partial_ref.at[pl.ds(0, ROWS), :],
            dst_ref=recv_ref.at[k],
            send_sem=rs_send.at[k],
            recv_sem=rs_recv.at[k],
            device_id=(me,),
            device_id_type=pl.DeviceIdType.MESH,
        )
        rcv.wait_recv()
        acc_ref[...] += recv_ref[k].astype(jnp.float32)

    red_ref[...] = acc_ref[...].astype(jnp.bfloat16)
    ag = []
    for k in range(1, N_DEV):
        dst = lax.rem(me + k, N_DEV)
        d = pltpu.make_async_remote_copy(
            src_ref=red_ref,
            dst_ref=out_ref.at[pl.ds(me * ROWS, ROWS), :],
            send_sem=ag_send.at[k],
            recv_sem=ag_recv.at[k],
            device_id=(dst,),
            device_id_type=pl.DeviceIdType.MESH,
        )
        d.start()
        ag.append(d)

    out_ref[pl.ds(me * ROWS, ROWS), :] = red_ref[...]

    for k in range(1, N_DEV):
        src = lax.rem(me - k + N_DEV, N_DEV)
        recv = pltpu.make_async_remote_copy(
            src_ref=red_ref,
            dst_ref=out_ref.at[pl.ds(src * ROWS, ROWS), :],
            send_sem=ag_send.at[k],
            recv_sem=ag_recv.at[k],
            device_id=(src,),
            device_id_type=pl.DeviceIdType.MESH,
        )
        recv.wait_recv()

    for k in range(1, N_DEV):
        snd = pltpu.make_async_remote_copy(
            src_ref=partial_ref.at[pl.ds(0, ROWS), :],
            dst_ref=recv_ref.at[k],
            send_sem=rs_send.at[k],
            recv_sem=rs_recv.at[k],
            device_id=(me,),
            device_id_type=pl.DeviceIdType.MESH,
        )
        snd.wait_send()
        ag[k - 1].wait_send()


def kernel(x, Wq, Wk, Wv, Wo):
    bf = jnp.bfloat16
    cos, sin, qcos, qsin, rot = _tables()
    out = pl.pallas_call(
        _fused_body,
        out_shape=jax.ShapeDtypeStruct((SQ, D), bf),
        in_specs=[pl.BlockSpec(memory_space=pltpu.VMEM)] * 10,
        out_specs=pl.BlockSpec(memory_space=pltpu.VMEM),
        scratch_shapes=[
            pltpu.VMEM((HQ, SQ, DH), bf),
            pltpu.VMEM((HQ, SQ, DH), bf),
            pltpu.VMEM((HQ, SQ, DH), bf),
            pltpu.VMEM((SQ, D), bf),
            pltpu.VMEM((N_DEV, ROWS, D), bf),
            pltpu.VMEM((ROWS, D), jnp.float32),
            pltpu.VMEM((ROWS, D), bf),
            pltpu.SemaphoreType.DMA((N_DEV,)),
            pltpu.SemaphoreType.DMA((N_DEV,)),
            pltpu.SemaphoreType.DMA((N_DEV,)),
            pltpu.SemaphoreType.DMA((N_DEV,)),
        ],
        compiler_params=(pltpu.CompilerParams(collective_id=0) if _COMM
                         else pltpu.CompilerParams()),
    )(x[0].astype(bf), Wq.astype(bf), Wk.astype(bf), Wv.astype(bf),
      Wo.astype(bf).reshape(HQ, DH, D), cos, sin, qcos, qsin, rot)
    return out[None, :, :]
